# baseline (speedup 1.0000x reference)
import functools

import jax
import jax.numpy as jnp
from jax import lax
from jax.experimental import pallas as pl
from jax.experimental.pallas import tpu as pltpu
from jax.experimental.pallas import tpu_sc as plsc

B = 16384
L = 50
D = 64
DW = D // 2
N_TOK = B * L

NW = 32
CH = 128
PER_W = N_TOK // NW
NCHUNK = PER_W // CH


def _make_kernel():
    mesh = plsc.VectorSubcoreMesh(core_axis_name="c", subcore_axis_name="s")

    @functools.partial(
        pl.kernel,
        mesh=mesh,
        compiler_params=pltpu.CompilerParams(use_tc_tiling_on_sc=False),
        out_type=jax.ShapeDtypeStruct((N_TOK, DW), jnp.int32),
        scratch_types=[
            pltpu.VMEM((NCHUNK, CH), jnp.int32),
            pltpu.VMEM((CH, DW), jnp.int32),
            pltpu.SemaphoreType.DMA,
        ],
    )
    def gather_kernel(tok_hbm, tab_hbm, out_hbm, idx_v, rows_v, sem):
        wid = lax.axis_index("s") * 2 + lax.axis_index("c")
        base = wid * PER_W
        pltpu.sync_copy(tok_hbm.at[pl.ds(wid * NCHUNK, NCHUNK)], idx_v)

        def body(j, carry):
            pltpu.async_copy(tab_hbm.at[idx_v.at[j]], rows_v, sem).wait()
            pltpu.sync_copy(rows_v, out_hbm.at[pl.ds(base + j * CH, CH)])
            return carry

        lax.fori_loop(0, NCHUNK, body, 0)

    return gather_kernel


_gather = _make_kernel()


def kernel(tokens, table):
    tok = tokens.reshape(NW * NCHUNK, CH)
    tab_i32 = jax.lax.bitcast_convert_type(
        table.reshape(table.shape[0], DW, 2), jnp.int32)
    out = _gather(tok, tab_i32)
    out_f16 = jax.lax.bitcast_convert_type(out, jnp.float16)
    return out_f16.reshape(1, B, L, D)


# speedup vs baseline: 1.8739x; 1.8739x over previous
import functools

import jax
import jax.numpy as jnp
from jax import lax
from jax.experimental import pallas as pl
from jax.experimental.pallas import tpu as pltpu
from jax.experimental.pallas import tpu_sc as plsc

B = 16384
L = 50
D = 64
DW = D // 2
V = 1000000
N_TOK = B * L

NW = 32
CH = 100
PER_W = N_TOK // NW
NCHUNK = PER_W // CH

TR = V // NW
RC = 625
RCH = TR // RC

SENTINEL = 1


def _make_kernel():
    mesh = plsc.VectorSubcoreMesh(core_axis_name="c", subcore_axis_name="s")

    @functools.partial(
        pl.kernel,
        mesh=mesh,
        compiler_params=pltpu.CompilerParams(
            use_tc_tiling_on_sc=False, needs_layout_passes=False),
        out_type=(
            jax.ShapeDtypeStruct((1, B, L, D), jnp.uint16),
            jax.ShapeDtypeStruct((NW, 128), jnp.int32),
        ),
        scratch_types=[
            pltpu.HBM((V, DW), jnp.int32),
            pltpu.VMEM((RC, D), jnp.uint16),
            pltpu.VMEM((RC, DW), jnp.int32),
            pltpu.VMEM((NCHUNK, CH), jnp.int32),
            pltpu.VMEM((CH, DW), jnp.int32),
            pltpu.VMEM((2, L, D), jnp.uint16),
            pltpu.VMEM((NW, 128), jnp.int32),
            pltpu.VMEM((1, 128), jnp.int32),
            pltpu.SemaphoreType.DMA,
        ],
    )
    def gather_kernel(tok_hbm, tab_hbm, out_hbm, flags_hbm,
                      tabw, u16_v, i32_v, idx_v, rows_v, outb_v,
                      flag_v, const_v, sem):
        wid = lax.axis_index("s") * 2 + lax.axis_index("c")

        for k in range(8):
            const_v[0, pl.ds(16 * k, 16)] = jnp.zeros((16,), jnp.int32)
        pltpu.sync_copy(const_v, flags_hbm.at[pl.ds(wid, 1)])

        def repack(c, carry):
            r0 = wid * TR + c * RC
            pltpu.sync_copy(tab_hbm.at[pl.ds(r0, RC)], u16_v)

            def conv(i, carry2):
                for u in range(10):
                    row = i * 5 + (u // 2)
                    h = u % 2
                    half = u16_v[row, pl.ds(32 * h, 32)]
                    i32_v[row, pl.ds(16 * h, 16)] = plsc.bitcast(
                        half, jnp.int32)
                return carry2

            lax.fori_loop(0, RC * 2 // 10, conv, 0)
            pltpu.sync_copy(i32_v, tabw.at[pl.ds(r0, RC)])
            return carry

        lax.fori_loop(0, RCH, repack, 0)

        for k in range(8):
            const_v[0, pl.ds(16 * k, 16)] = jnp.full(
                (16,), SENTINEL, jnp.int32)
        pltpu.sync_copy(const_v, flags_hbm.at[pl.ds(wid, 1)])

        ones16 = jnp.full((16,), 1, jnp.int32)
        zeros16 = jnp.zeros((16,), jnp.int32)

        def spin_cond(n):
            return n != NW * 128

        def spin_body(n):
            pltpu.sync_copy(flags_hbm, flag_v)
            acc = zeros16

            def row_acc(r, a):
                for k in range(8):
                    chunk = flag_v[r, pl.ds(16 * k, 16)]
                    a = a + jnp.where(chunk == SENTINEL, ones16, zeros16)
                return a

            acc = lax.fori_loop(0, NW, row_acc, acc)
            return jnp.sum(acc)

        lax.while_loop(spin_cond, spin_body, jnp.int32(0))

        pltpu.sync_copy(tok_hbm.at[pl.ds(wid * NCHUNK, NCHUNK)], idx_v)

        def gather_chunk(j, carry):
            pltpu.async_copy(tabw.at[idx_v.at[j]], rows_v, sem).wait()

            def conv(i, carry2):
                for u in range(10):
                    tok_i = i * 5 + (u // 2)
                    h = u % 2
                    w16 = plsc.bitcast(
                        rows_v[tok_i, pl.ds(16 * h, 16)], jnp.uint16)
                    outb_v[tok_i // L, tok_i % L, pl.ds(32 * h, 32)] = w16
                return carry2

            lax.fori_loop(0, CH * 2 // 10, conv, 0)
            b0 = wid * (B // NW) + 2 * j
            pltpu.sync_copy(outb_v, out_hbm.at[0, pl.ds(b0, 2)])
            return carry

        lax.fori_loop(0, NCHUNK, gather_chunk, 0)

    return gather_kernel


_gather = _make_kernel()


def kernel(tokens, table):
    tok = tokens.reshape(NW * NCHUNK, CH)
    tab_u16 = lax.bitcast_convert_type(table, jnp.uint16)
    out_u16, _ = _gather(tok, tab_u16)
    return lax.bitcast_convert_type(out_u16, jnp.float16)


# speedup vs baseline: 2.0455x; 1.0916x over previous
import functools

import jax
import jax.numpy as jnp
from jax import lax
from jax.experimental import pallas as pl
from jax.experimental.pallas import tpu as pltpu
from jax.experimental.pallas import tpu_sc as plsc

B = 16384
L = 50
D = 64
DW = D // 2
V = 1000000
N_TOK = B * L

NW = 32
CH = 100
PER_W = N_TOK // NW
NCHUNK = PER_W // CH

TR = V // NW
RC = 625
RCH = TR // RC

SENTINEL = 1


def _make_kernel():
    mesh = plsc.VectorSubcoreMesh(core_axis_name="c", subcore_axis_name="s")

    @functools.partial(
        pl.kernel,
        mesh=mesh,
        compiler_params=pltpu.CompilerParams(
            use_tc_tiling_on_sc=False, needs_layout_passes=False),
        out_type=(
            jax.ShapeDtypeStruct((1, B, L, D), jnp.float16),
            jax.ShapeDtypeStruct((NW, 128), jnp.int32),
        ),
        scratch_types=[
            pltpu.HBM((V, DW), jnp.int32),
            pltpu.VMEM((RC, D), jnp.float16),
            pltpu.VMEM((RC, DW), jnp.int32),
            pltpu.VMEM((NCHUNK, CH), jnp.int32),
            pltpu.VMEM((CH, DW), jnp.int32),
            pltpu.VMEM((2, L, D), jnp.float16),
            pltpu.VMEM((NW, 128), jnp.int32),
            pltpu.VMEM((1, 128), jnp.int32),
            pltpu.SemaphoreType.DMA,
        ],
    )
    def gather_kernel(tok_hbm, tab_hbm, out_hbm, flags_hbm,
                      tabw, f16_v, i32_v, idx_v, rows_v, outb_v,
                      flag_v, const_v, sem):
        wid = lax.axis_index("s") * 2 + lax.axis_index("c")

        for k in range(8):
            const_v[0, pl.ds(16 * k, 16)] = jnp.zeros((16,), jnp.int32)
        pltpu.sync_copy(const_v, flags_hbm.at[pl.ds(wid, 1)])

        def repack(c, carry):
            r0 = wid * TR + c * RC
            pltpu.sync_copy(tab_hbm.at[pl.ds(r0, RC)], f16_v)

            def conv(i, carry2):
                for u in range(10):
                    row = i * 5 + (u // 2)
                    h = u % 2
                    half = f16_v[row, pl.ds(32 * h, 32)]
                    i32_v[row, pl.ds(16 * h, 16)] = plsc.bitcast(
                        half, jnp.int32)
                return carry2

            lax.fori_loop(0, RC * 2 // 10, conv, 0)
            pltpu.sync_copy(i32_v, tabw.at[pl.ds(r0, RC)])
            return carry

        lax.fori_loop(0, RCH, repack, 0)

        for k in range(8):
            const_v[0, pl.ds(16 * k, 16)] = jnp.full(
                (16,), SENTINEL, jnp.int32)
        pltpu.sync_copy(const_v, flags_hbm.at[pl.ds(wid, 1)])

        ones16 = jnp.full((16,), 1, jnp.int32)
        zeros16 = jnp.zeros((16,), jnp.int32)

        def spin_cond(n):
            return n != NW * 128

        def spin_body(n):
            pltpu.sync_copy(flags_hbm, flag_v)
            acc = zeros16

            def row_acc(r, a):
                for k in range(8):
                    chunk = flag_v[r, pl.ds(16 * k, 16)]
                    a = a + jnp.where(chunk == SENTINEL, ones16, zeros16)
                return a

            acc = lax.fori_loop(0, NW, row_acc, acc)
            return jnp.sum(acc)

        lax.while_loop(spin_cond, spin_body, jnp.int32(0))

        pltpu.sync_copy(tok_hbm.at[pl.ds(wid * NCHUNK, NCHUNK)], idx_v)

        def gather_chunk(j, carry):
            pltpu.async_copy(tabw.at[idx_v.at[j]], rows_v, sem).wait()

            def conv(i, carry2):
                for u in range(10):
                    tok_i = i * 5 + (u // 2)
                    h = u % 2
                    w16 = plsc.bitcast(
                        rows_v[tok_i, pl.ds(16 * h, 16)], jnp.float16)
                    outb_v[tok_i // L, tok_i % L, pl.ds(32 * h, 32)] = w16
                return carry2

            lax.fori_loop(0, CH * 2 // 10, conv, 0)
            b0 = wid * (B // NW) + 2 * j
            pltpu.sync_copy(outb_v, out_hbm.at[0, pl.ds(b0, 2)])
            return carry

        lax.fori_loop(0, NCHUNK, gather_chunk, 0)

    return gather_kernel


_gather = _make_kernel()


def kernel(tokens, table):
    tok = tokens.reshape(NW * NCHUNK, CH)
    out, _ = _gather(tok, table)
    return out


# speedup vs baseline: 2.3439x; 1.1459x over previous
import functools

import jax
import jax.numpy as jnp
from jax import lax
from jax.experimental import pallas as pl
from jax.experimental.pallas import tpu as pltpu
from jax.experimental.pallas import tpu_sc as plsc

B = 16384
L = 50
D = 64
DW = D // 2
V = 1000000
N_TOK = B * L

NW = 32
CH = 100
PER_W = N_TOK // NW
NCHUNK = PER_W // CH

TR = V // NW
RC = 625
RCH = TR // RC

SENTINEL = 1


def _make_kernel():
    mesh = plsc.VectorSubcoreMesh(core_axis_name="c", subcore_axis_name="s")

    @functools.partial(
        pl.kernel,
        mesh=mesh,
        compiler_params=pltpu.CompilerParams(
            use_tc_tiling_on_sc=False, needs_layout_passes=False),
        out_type=(
            jax.ShapeDtypeStruct((1, B, L, D), jnp.float16),
            jax.ShapeDtypeStruct((NW, 128), jnp.int32),
        ),
        scratch_types=[
            pltpu.HBM((V, DW), jnp.int32),
            pltpu.VMEM((RC, D), jnp.float16),
            pltpu.VMEM((RC, D), jnp.float16),
            pltpu.VMEM((RC, DW), jnp.int32),
            pltpu.VMEM((NCHUNK, CH), jnp.int32),
            pltpu.VMEM((CH, DW), jnp.int32),
            pltpu.VMEM((CH, DW), jnp.int32),
            pltpu.VMEM((2, L, D), jnp.float16),
            pltpu.VMEM((NW, 128), jnp.int32),
            pltpu.VMEM((1, 128), jnp.int32),
            pltpu.SemaphoreType.DMA,
            pltpu.SemaphoreType.DMA,
        ],
    )
    def gather_kernel(tok_hbm, tab_hbm, out_hbm, flags_hbm,
                      tabw, f16_a, f16_b, i32_v, idx_v, rows_a, rows_b,
                      outb_v, flag_v, const_v, sem_a, sem_b):
        wid = lax.axis_index("s") * 2 + lax.axis_index("c")

        for k in range(8):
            const_v[0, pl.ds(16 * k, 16)] = jnp.zeros((16,), jnp.int32)
        pltpu.sync_copy(const_v, flags_hbm.at[pl.ds(wid, 1)])

        rbase = wid * TR

        def rconv(buf):
            def conv(i, carry2):
                for u in range(10):
                    row = i * 5 + (u // 2)
                    h = u % 2
                    half = buf[row, pl.ds(32 * h, 32)]
                    i32_v[row, pl.ds(16 * h, 16)] = plsc.bitcast(
                        half, jnp.int32)
                return carry2
            lax.fori_loop(0, RC * 2 // 10, conv, 0)

        pltpu.async_copy(tab_hbm.at[pl.ds(rbase, RC)], f16_a, sem_a)

        def repack_pair(cc, carry):
            c0 = 2 * cc
            pltpu.async_copy(
                tab_hbm.at[pl.ds(rbase + (c0 + 1) * RC, RC)], f16_b, sem_b)
            pltpu.make_async_copy(
                tab_hbm.at[pl.ds(rbase, RC)], f16_a, sem_a).wait()
            rconv(f16_a)
            pltpu.sync_copy(i32_v, tabw.at[pl.ds(rbase + c0 * RC, RC)])

            @pl.when(cc < RCH // 2 - 1)
            def _():
                pltpu.async_copy(
                    tab_hbm.at[pl.ds(rbase + (c0 + 2) * RC, RC)],
                    f16_a, sem_a)

            pltpu.make_async_copy(
                tab_hbm.at[pl.ds(rbase, RC)], f16_b, sem_b).wait()
            rconv(f16_b)
            pltpu.sync_copy(i32_v, tabw.at[pl.ds(rbase + (c0 + 1) * RC, RC)])
            return carry

        lax.fori_loop(0, RCH // 2, repack_pair, 0)

        for k in range(8):
            const_v[0, pl.ds(16 * k, 16)] = jnp.full(
                (16,), SENTINEL, jnp.int32)
        pltpu.sync_copy(const_v, flags_hbm.at[pl.ds(wid, 1)])

        ones16 = jnp.full((16,), 1, jnp.int32)
        zeros16 = jnp.zeros((16,), jnp.int32)

        def spin_cond(n):
            return n != NW * 128

        def spin_body(n):
            pltpu.sync_copy(flags_hbm, flag_v)
            acc = zeros16

            def row_acc(r, a):
                for k in range(8):
                    chunk = flag_v[r, pl.ds(16 * k, 16)]
                    a = a + jnp.where(chunk == SENTINEL, ones16, zeros16)
                return a

            acc = lax.fori_loop(0, NW, row_acc, acc)
            return jnp.sum(acc)

        lax.while_loop(spin_cond, spin_body, jnp.int32(0))

        pltpu.sync_copy(tok_hbm.at[pl.ds(wid * NCHUNK, NCHUNK)], idx_v)
        bbase = wid * (B // NW)

        def gconv(buf, j):
            def conv(i, carry2):
                for u in range(10):
                    tok_i = i * 5 + (u // 2)
                    h = u % 2
                    w16 = plsc.bitcast(
                        buf[tok_i, pl.ds(16 * h, 16)], jnp.float16)
                    outb_v[tok_i // L, tok_i % L, pl.ds(32 * h, 32)] = w16
                return carry2
            lax.fori_loop(0, CH * 2 // 10, conv, 0)
            pltpu.sync_copy(outb_v, out_hbm.at[0, pl.ds(bbase + 2 * j, 2)])

        pltpu.async_copy(tabw.at[idx_v.at[0]], rows_a, sem_a)

        def gather_pair(jj, carry):
            j0 = 2 * jj
            pltpu.async_copy(tabw.at[idx_v.at[j0 + 1]], rows_b, sem_b)
            pltpu.make_async_copy(
                tabw.at[pl.ds(0, CH)], rows_a, sem_a).wait()
            gconv(rows_a, j0)

            @pl.when(jj < NCHUNK // 2 - 1)
            def _():
                pltpu.async_copy(tabw.at[idx_v.at[j0 + 2]], rows_a, sem_a)

            pltpu.make_async_copy(
                tabw.at[pl.ds(0, CH)], rows_b, sem_b).wait()
            gconv(rows_b, j0 + 1)
            return carry

        lax.fori_loop(0, NCHUNK // 2, gather_pair, 0)

    return gather_kernel


_gather = _make_kernel()


def kernel(tokens, table):
    tok = tokens.reshape(NW * NCHUNK, CH)
    out, _ = _gather(tok, table)
    return out
